# baseline (speedup 1.0000x reference)
import functools

import jax
import jax.numpy as jnp
from jax import lax
from jax.experimental import pallas as pl
from jax.experimental.pallas import tpu as pltpu
from jax.experimental.pallas import tpu_sc as plsc

_SEQ = 8192
_C = 13
_B = 4
_NUM_CORES = 2
_NUM_SUBCORES = 16
_NW = _NUM_CORES * _NUM_SUBCORES
_CHUNK = _SEQ // _NW
_L = 16


@functools.partial(
    pl.kernel,
    mesh=plsc.VectorSubcoreMesh(core_axis_name="c", subcore_axis_name="s"),
    out_type=jax.ShapeDtypeStruct((_C, _SEQ), jnp.float32),
    compiler_params=pltpu.CompilerParams(needs_layout_passes=False),
    scratch_types=[
        pltpu.VMEM((_CHUNK, _C), jnp.float32),
        pltpu.VMEM((_C, _CHUNK), jnp.float32),
        pltpu.SemaphoreType.DMA,
    ],
)
def _transpose_sc(w_hbm, out_hbm, w_tile, out_tile, sem):
    wid = lax.axis_index("s") * _NUM_CORES + lax.axis_index("c")
    base = wid * _CHUNK
    pltpu.sync_copy(w_hbm.at[pl.ds(base, _CHUNK), :], w_tile)
    lane = lax.iota(jnp.int32, _L)
    copies = []
    for c0 in range(0, _C, 4):
        nc = min(4, _C - c0)
        for c in range(c0, c0 + nc):
            col = jnp.full((_L,), c, jnp.int32)
            vals = [
                plsc.load_gather(w_tile, [lane + jb, col])
                for jb in range(0, _CHUNK, _L)
            ]
            for i, jb in enumerate(range(0, _CHUNK, _L)):
                out_tile[c, pl.ds(jb, _L)] = vals[i]
        cp = pltpu.make_async_copy(
            out_tile.at[pl.ds(c0, nc), :],
            out_hbm.at[pl.ds(c0, nc), pl.ds(base, _CHUNK)],
            sem,
        )
        cp.start()
        copies.append(cp)
    for cp in copies:
        cp.wait()


def _broadcast_body(wt_ref, out_ref):
    out_ref[0] = wt_ref[...]


_broadcast_tc = pl.pallas_call(
    _broadcast_body,
    grid=(_B,),
    in_specs=[pl.BlockSpec((_C, _SEQ), lambda b: (0, 0))],
    out_specs=pl.BlockSpec((1, _C, _SEQ), lambda b: (b, 0, 0)),
    out_shape=jax.ShapeDtypeStruct((_B, _C, _SEQ), jnp.float32),
)


def kernel(x, row_embed_weight):
    del x
    wt = _transpose_sc(row_embed_weight)
    return _broadcast_tc(wt)


# speedup vs baseline: 1.4073x; 1.4073x over previous
import functools

import jax
import jax.numpy as jnp
from jax import lax
from jax.experimental import pallas as pl
from jax.experimental.pallas import tpu as pltpu
from jax.experimental.pallas import tpu_sc as plsc

_SEQ = 8192
_C = 13
_B = 4
_LANES = 128
_NBLK = _SEQ // _LANES
_NUM_CORES = 2
_NUM_SUBCORES = 16
_NW = _NUM_CORES * _NUM_SUBCORES
_BLK_PER_W = _NBLK // _NW
_CHUNK = _BLK_PER_W * _LANES
_L = 16


@functools.partial(
    pl.kernel,
    mesh=plsc.VectorSubcoreMesh(core_axis_name="c", subcore_axis_name="s"),
    out_type=jax.ShapeDtypeStruct((_C, _NBLK, _B, _LANES), jnp.float32),
    compiler_params=pltpu.CompilerParams(needs_layout_passes=False),
    scratch_types=[
        pltpu.VMEM((_C, _CHUNK), jnp.float32),
        pltpu.VMEM((_C, _BLK_PER_W, _B, _LANES), jnp.float32),
        pltpu.SemaphoreType.DMA,
    ],
)
def _bcast_sc(wt_hbm, out_hbm, staged, rep, sem):
    wid = lax.axis_index("s") * _NUM_CORES + lax.axis_index("c")
    base = wid * _CHUNK
    pltpu.sync_copy(wt_hbm.at[:, pl.ds(base, _CHUNK)], staged)
    copies = []
    for c in range(_C):
        for j in range(_BLK_PER_W):
            vals = [
                staged[c, pl.ds(j * _LANES + k * _L, _L)]
                for k in range(_LANES // _L)
            ]
            for b in range(_B):
                for k in range(_LANES // _L):
                    rep[c, j, b, pl.ds(k * _L, _L)] = vals[k]
        cp = pltpu.make_async_copy(
            rep.at[c],
            out_hbm.at[c, pl.ds(wid * _BLK_PER_W, _BLK_PER_W), :, :],
            sem,
        )
        cp.start()
        copies.append(cp)
    for cp in copies:
        cp.wait()


def kernel(x, row_embed_weight):
    del x
    wt = row_embed_weight.T
    tmp = _bcast_sc(wt)
    return jnp.transpose(tmp, (2, 0, 1, 3)).reshape(_B, _C, _SEQ)


# speedup vs baseline: 1.5452x; 1.0980x over previous
import functools

import jax
import jax.numpy as jnp
from jax import lax
from jax.experimental import pallas as pl
from jax.experimental.pallas import tpu as pltpu
from jax.experimental.pallas import tpu_sc as plsc

_SEQ = 8192
_C = 13
_B = 4
_LANES = 128
_NBLK = _SEQ // _LANES
_NUM_CORES = 2
_NUM_SUBCORES = 16
_NW = _NUM_CORES * _NUM_SUBCORES
_HALF = _SEQ // 2
_HALF_BLK = _NBLK // 2


@functools.partial(
    pl.kernel,
    mesh=plsc.VectorSubcoreMesh(core_axis_name="c", subcore_axis_name="s"),
    out_type=jax.ShapeDtypeStruct((_C, _NBLK, _B, _LANES), jnp.float32),
    compiler_params=pltpu.CompilerParams(needs_layout_passes=False),
    scratch_types=[
        pltpu.VMEM((_HALF_BLK, _LANES), jnp.float32),
        pltpu.SemaphoreType.DMA,
    ],
)
def _bcast_sc(wt_hbm, out_hbm, staged, sem):
    wid = lax.axis_index("s") * _NUM_CORES + lax.axis_index("c")
    c = wid // 2
    h = wid % 2

    @pl.when(c < _C)
    def _():
        pltpu.sync_copy(
            wt_hbm.at[pl.ds(c, 1), pl.ds(h * _HALF, _HALF)],
            staged.reshape(1, _HALF),
        )
        copies = []
        for b in range(_B):
            cp = pltpu.make_async_copy(
                staged,
                out_hbm.at[c, pl.ds(h * _HALF_BLK, _HALF_BLK), b, :],
                sem,
            )
            cp.start()
            copies.append(cp)
        for cp in copies:
            cp.wait()


def kernel(x, row_embed_weight):
    del x
    wt = row_embed_weight.T
    tmp = _bcast_sc(wt)
    return jnp.transpose(tmp, (2, 0, 1, 3)).reshape(_B, _C, _SEQ)
